# baseline (speedup 1.0000x reference)
import functools

import jax
import jax.numpy as jnp
from jax import lax
from jax.experimental import pallas as pl
from jax.experimental.pallas import tpu as pltpu
from jax.experimental.pallas import tpu_sc as plsc

HIDDEN = 16
NUM_WORKERS = 32
CHUNK = 1600


NBUF = 3


def _gather_body(ids_hbm, table_hbm, out_hbm,
                 idx_a, idx_b, idx_c, rows_a, rows_b, rows_c,
                 gsem_a, gsem_b, gsem_c, ssem_a, ssem_b, ssem_c,
                 *, rows_per_worker, n_chunks):
    wid = lax.axis_index("s") * 2 + lax.axis_index("c")
    base = wid * rows_per_worker

    idx = (idx_a, idx_b, idx_c)
    rows = (rows_a, rows_b, rows_c)
    gsems = (gsem_a, gsem_b, gsem_c)
    ssems = (ssem_a, ssem_b, ssem_c)

    gather_cp = [None] * NBUF
    store_cp = [None] * NBUF
    prev = None
    for j in range(n_chunks):
        s = j % NBUF
        off = base + j * CHUNK
        if store_cp[s] is not None:
            store_cp[s].wait()
            store_cp[s] = None
        pltpu.sync_copy(ids_hbm.at[pl.ds(off, CHUNK)], idx[s])
        gather_cp[s] = pltpu.async_copy(table_hbm.at[idx[s]], rows[s], gsems[s])
        if prev is not None:
            ps, poff = prev
            gather_cp[ps].wait()
            store_cp[ps] = pltpu.async_copy(
                rows[ps], out_hbm.at[pl.ds(poff, CHUNK), pl.ds(0, HIDDEN)],
                ssems[ps])
        prev = (s, off)
    ps, poff = prev
    gather_cp[ps].wait()
    store_cp[ps] = pltpu.async_copy(
        rows[ps], out_hbm.at[pl.ds(poff, CHUNK), pl.ds(0, HIDDEN)], ssems[ps])
    for s in range(NBUF):
        if store_cp[s] is not None:
            store_cp[s].wait()


def kernel(input_ids, attention_mask, emb_weight):
    del attention_mask
    batch, seq = input_ids.shape
    vocab = emb_weight.shape[0]
    total = batch * seq
    rows_per_worker = total // NUM_WORKERS
    n_chunks = rows_per_worker // CHUNK

    flat_ids = input_ids.reshape(total).astype(jnp.int32)

    table = emb_weight

    mesh = plsc.VectorSubcoreMesh(core_axis_name="c", subcore_axis_name="s")

    def _ids_body(src_hbm, dst_hbm, buf):
        wid = lax.axis_index("s") * 2 + lax.axis_index("c")
        off = wid * rows_per_worker
        pltpu.sync_copy(src_hbm.at[pl.ds(off, rows_per_worker)], buf)
        pltpu.sync_copy(buf, dst_hbm.at[pl.ds(off, rows_per_worker)])

    flat_ids = pl.kernel(
        _ids_body,
        out_type=jax.ShapeDtypeStruct((total,), jnp.int32),
        mesh=mesh,
        scratch_types=[pltpu.VMEM((rows_per_worker,), jnp.int32)],
        compiler_params=pltpu.CompilerParams(use_tc_tiling_on_sc=False),
    )(flat_ids)
    out2d = pl.kernel(
        functools.partial(_gather_body, rows_per_worker=rows_per_worker,
                          n_chunks=n_chunks),
        out_type=jax.ShapeDtypeStruct((total, 128), jnp.float32),
        mesh=mesh,
        scratch_types=(
            [pltpu.VMEM((CHUNK,), jnp.int32)] * NBUF
            + [pltpu.VMEM((CHUNK, HIDDEN), jnp.float32)] * NBUF
            + [pltpu.SemaphoreType.DMA] * (2 * NBUF)
        ),
        compiler_params=pltpu.CompilerParams(use_tc_tiling_on_sc=False),
    )(flat_ids, table)

    return out2d.reshape(batch, seq, 128)[:, :, :HIDDEN]


# speedup vs baseline: 1.0008x; 1.0008x over previous
import functools

import jax
import jax.numpy as jnp
from jax import lax
from jax.experimental import pallas as pl
from jax.experimental.pallas import tpu as pltpu
from jax.experimental.pallas import tpu_sc as plsc

HIDDEN = 16
NUM_WORKERS = 32
CHUNK = 1600


NBUF = 3


def _gather_body(ids_hbm, table_hbm, out_hbm,
                 idx_a, idx_b, idx_c, rows_a, rows_b, rows_c,
                 gsem_a, gsem_b, gsem_c, ssem_a, ssem_b, ssem_c,
                 *, rows_per_worker, n_chunks):
    wid = lax.axis_index("s") * 2 + lax.axis_index("c")
    base = wid * rows_per_worker

    idx = (idx_a, idx_b, idx_c)
    rows = (rows_a, rows_b, rows_c)
    gsems = (gsem_a, gsem_b, gsem_c)
    ssems = (ssem_a, ssem_b, ssem_c)

    gather_cp = [None] * NBUF
    store_cp = [None] * NBUF
    prev = None
    for j in range(n_chunks):
        s = j % NBUF
        off = base + j * CHUNK
        if store_cp[s] is not None:
            store_cp[s].wait()
            store_cp[s] = None
        pltpu.sync_copy(ids_hbm.at[pl.ds(off, CHUNK)], idx[s])
        gather_cp[s] = pltpu.async_copy(table_hbm.at[idx[s]], rows[s], gsems[s])
        if prev is not None:
            ps, poff = prev
            gather_cp[ps].wait()
            store_cp[ps] = pltpu.async_copy(
                rows[ps], out_hbm.at[pl.ds(poff, CHUNK), pl.ds(0, HIDDEN)],
                ssems[ps])
        prev = (s, off)
    ps, poff = prev
    gather_cp[ps].wait()
    store_cp[ps] = pltpu.async_copy(
        rows[ps], out_hbm.at[pl.ds(poff, CHUNK), pl.ds(0, HIDDEN)], ssems[ps])
    for s in range(NBUF):
        if store_cp[s] is not None:
            store_cp[s].wait()


def kernel(input_ids, attention_mask, emb_weight):
    del attention_mask
    batch, seq = input_ids.shape
    vocab = emb_weight.shape[0]
    total = batch * seq
    rows_per_worker = total // NUM_WORKERS
    n_chunks = rows_per_worker // CHUNK

    flat_ids = input_ids.reshape(total).astype(jnp.int32)

    table = emb_weight

    mesh = plsc.VectorSubcoreMesh(core_axis_name="c", subcore_axis_name="s")
    out2d = pl.kernel(
        functools.partial(_gather_body, rows_per_worker=rows_per_worker,
                          n_chunks=n_chunks),
        out_type=jax.ShapeDtypeStruct((total, 128), jnp.float32),
        mesh=mesh,
        scratch_types=(
            [pltpu.VMEM((CHUNK,), jnp.int32)] * NBUF
            + [pltpu.VMEM((CHUNK, HIDDEN), jnp.float32)] * NBUF
            + [pltpu.SemaphoreType.DMA] * (2 * NBUF)
        ),
        compiler_params=pltpu.CompilerParams(use_tc_tiling_on_sc=False),
    )(flat_ids, table)

    return out2d.reshape(batch, seq, 128)[:, :, :HIDDEN]
